# baseline (speedup 1.0000x reference)
import jax
import jax.numpy as jnp
from jax.experimental import pallas as pl
from jax.experimental.pallas import tpu as pltpu

EPS = 1e-3
FORGET_BIAS = 1.0


def _ln(v, gamma, beta):
    mean = jnp.mean(v, axis=1, keepdims=True)
    vc = v - mean
    var = jnp.mean(vc * vc, axis=1, keepdims=True)
    return gamma * (vc * jax.lax.rsqrt(var + EPS)) + beta


def _lstm_kernel(x_ref, c_ref, h_ref, wx_ref, wh_ref, b_ref, g_ref, be_ref,
                 gc_ref, bc_ref, h_out_ref, c_out_ref, gates_ref):
    H = c_ref.shape[1]
    s = pl.program_id(0)
    slot = jax.lax.rem(s, 2)

    x = x_ref[...]
    h = h_ref[...]
    for g in range(4):
        sl = slice(g * H, (g + 1) * H)
        acc = jnp.dot(x, wx_ref[:, sl], preferred_element_type=jnp.float32)
        acc = acc + jnp.dot(h, wh_ref[:, sl], preferred_element_type=jnp.float32)
        acc = acc + b_ref[:, sl]
        gates_ref[slot, :, sl] = _ln(acc, g_ref[:, sl], be_ref[:, sl])

    prev = 1 - slot
    gi = gates_ref[prev, :, 0:H]
    gj = gates_ref[prev, :, H:2 * H]
    gf = gates_ref[prev, :, 2 * H:3 * H]
    go = gates_ref[prev, :, 3 * H:4 * H]
    c = c_ref[...]
    new_c = c * jax.nn.sigmoid(gf + FORGET_BIAS) + jax.nn.sigmoid(gi) * jnp.tanh(gj)
    c_out_ref[...] = new_c
    c_ln = _ln(new_c, gc_ref[...], bc_ref[...])
    h_out_ref[...] = jnp.tanh(c_ln) * jax.nn.sigmoid(go)


def kernel(x, c, h, W_xh, W_hh, bias, ln_gamma, ln_beta, ln_c_gamma, ln_c_beta):
    B, I = x.shape
    H = c.shape[1]
    BB = min(256, B)
    nb = B // BB

    b2 = bias.reshape(1, 4 * H)
    g2 = ln_gamma.reshape(1, 4 * H)
    be2 = ln_beta.reshape(1, 4 * H)
    gc2 = ln_c_gamma.reshape(1, H)
    bc2 = ln_c_beta.reshape(1, H)

    lead = lambda s: (jnp.minimum(s, nb - 1), 0)
    lag = lambda s: (jnp.maximum(s - 1, 0), 0)
    fixed = lambda s: (0, 0)
    new_h, new_c = pl.pallas_call(
        _lstm_kernel,
        grid=(nb + 1,),
        in_specs=[
            pl.BlockSpec((BB, I), lead),
            pl.BlockSpec((BB, H), lag),
            pl.BlockSpec((BB, H), lead),
            pl.BlockSpec((I, 4 * H), fixed),
            pl.BlockSpec((H, 4 * H), fixed),
            pl.BlockSpec((1, 4 * H), fixed),
            pl.BlockSpec((1, 4 * H), fixed),
            pl.BlockSpec((1, 4 * H), fixed),
            pl.BlockSpec((1, H), fixed),
            pl.BlockSpec((1, H), fixed),
        ],
        out_specs=[
            pl.BlockSpec((BB, H), lag),
            pl.BlockSpec((BB, H), lag),
        ],
        out_shape=[
            jax.ShapeDtypeStruct((B, H), jnp.float32),
            jax.ShapeDtypeStruct((B, H), jnp.float32),
        ],
        scratch_shapes=[pltpu.VMEM((2, BB, 4 * H), jnp.float32)],
        compiler_params=pltpu.CompilerParams(
            dimension_semantics=("arbitrary",),
            vmem_limit_bytes=100 * 1024 * 1024,
        ),
        name="lstm_cell_fused",
    )(x, c, h, W_xh, W_hh, b2, g2, be2, gc2, bc2)
    return new_h, new_c


# speedup vs baseline: 1.0386x; 1.0386x over previous
import jax
import jax.numpy as jnp
from jax.experimental import pallas as pl
from jax.experimental.pallas import tpu as pltpu

EPS = 1e-3
FORGET_BIAS = 1.0


def _ln(v, gamma, beta):
    mean = jnp.mean(v, axis=1, keepdims=True)
    vc = v - mean
    var = jnp.mean(vc * vc, axis=1, keepdims=True)
    return gamma * (vc * jax.lax.rsqrt(var + EPS)) + beta


def _lstm_kernel(x_ref, c_ref, h_ref, wx_ref, wh_ref, b_ref, g_ref, be_ref,
                 gc_ref, bc_ref, h_out_ref, c_out_ref, gates_ref):
    H = c_ref.shape[1]
    s = pl.program_id(0)
    slot = jax.lax.rem(s, 2)
    prev = 1 - slot

    gi = gates_ref[prev, :, 0:H]
    gj = gates_ref[prev, :, H:2 * H]
    gf = gates_ref[prev, :, 2 * H:3 * H]
    go = gates_ref[prev, :, 3 * H:4 * H]
    c = c_ref[...]
    new_c = c * jax.nn.sigmoid(gf + FORGET_BIAS) + jax.nn.sigmoid(gi) * jnp.tanh(gj)
    c_out_ref[...] = new_c
    c_ln = _ln(new_c, gc_ref[...], bc_ref[...])
    h_out_ref[...] = jnp.tanh(c_ln) * jax.nn.sigmoid(go)

    x = x_ref[...]
    h = h_ref[...]
    for g in range(4):
        sl = slice(g * H, (g + 1) * H)
        acc = jnp.dot(x, wx_ref[:, sl], preferred_element_type=jnp.float32)
        acc = acc + jnp.dot(h, wh_ref[:, sl], preferred_element_type=jnp.float32)
        acc = acc + b_ref[:, sl]
        gates_ref[slot, :, sl] = _ln(acc, g_ref[:, sl], be_ref[:, sl])


def kernel(x, c, h, W_xh, W_hh, bias, ln_gamma, ln_beta, ln_c_gamma, ln_c_beta):
    B, I = x.shape
    H = c.shape[1]
    BB = min(256, B)
    nb = B // BB

    b2 = bias.reshape(1, 4 * H)
    g2 = ln_gamma.reshape(1, 4 * H)
    be2 = ln_beta.reshape(1, 4 * H)
    gc2 = ln_c_gamma.reshape(1, H)
    bc2 = ln_c_beta.reshape(1, H)

    lead = lambda s: (jnp.minimum(s, nb - 1), 0)
    lag = lambda s: (jnp.maximum(s - 1, 0), 0)
    fixed = lambda s: (0, 0)
    new_h, new_c = pl.pallas_call(
        _lstm_kernel,
        grid=(nb + 1,),
        in_specs=[
            pl.BlockSpec((BB, I), lead),
            pl.BlockSpec((BB, H), lag),
            pl.BlockSpec((BB, H), lead),
            pl.BlockSpec((I, 4 * H), fixed),
            pl.BlockSpec((H, 4 * H), fixed),
            pl.BlockSpec((1, 4 * H), fixed),
            pl.BlockSpec((1, 4 * H), fixed),
            pl.BlockSpec((1, 4 * H), fixed),
            pl.BlockSpec((1, H), fixed),
            pl.BlockSpec((1, H), fixed),
        ],
        out_specs=[
            pl.BlockSpec((BB, H), lag),
            pl.BlockSpec((BB, H), lag),
        ],
        out_shape=[
            jax.ShapeDtypeStruct((B, H), jnp.float32),
            jax.ShapeDtypeStruct((B, H), jnp.float32),
        ],
        scratch_shapes=[pltpu.VMEM((2, BB, 4 * H), jnp.float32)],
        compiler_params=pltpu.CompilerParams(
            dimension_semantics=("arbitrary",),
            vmem_limit_bytes=100 * 1024 * 1024,
        ),
        name="lstm_cell_fused",
    )(x, c, h, W_xh, W_hh, b2, g2, be2, gc2, bc2)
    return new_h, new_c


# speedup vs baseline: 1.1278x; 1.0858x over previous
import jax
import jax.numpy as jnp
from jax.experimental import pallas as pl
from jax.experimental.pallas import tpu as pltpu

EPS = 1e-3
FORGET_BIAS = 1.0


def _ln(v, gamma, beta):
    mean = jnp.mean(v, axis=1, keepdims=True)
    vc = v - mean
    var = jnp.mean(vc * vc, axis=1, keepdims=True)
    return gamma * (vc * jax.lax.rsqrt(var + EPS)) + beta


def _lstm_kernel(x_ref, c_ref, h_ref, wx_ref, wh_ref, b_ref, g_ref, be_ref,
                 gc_ref, bc_ref, h_out_ref, c_out_ref):
    H = c_ref.shape[1]
    x = x_ref[...]
    h = h_ref[...]
    gates = []
    for g in range(4):
        sl = slice(g * H, (g + 1) * H)
        acc = jnp.dot(x, wx_ref[:, sl], preferred_element_type=jnp.float32)
        acc = acc + jnp.dot(h, wh_ref[:, sl], preferred_element_type=jnp.float32)
        acc = acc + b_ref[:, sl]
        gates.append(_ln(acc, g_ref[:, sl], be_ref[:, sl]))
    gi, gj, gf, go = gates
    c = c_ref[...]
    new_c = c * jax.nn.sigmoid(gf + FORGET_BIAS) + jax.nn.sigmoid(gi) * jnp.tanh(gj)
    c_out_ref[...] = new_c
    c_ln = _ln(new_c, gc_ref[...], bc_ref[...])
    h_out_ref[...] = jnp.tanh(c_ln) * jax.nn.sigmoid(go)


def kernel(x, c, h, W_xh, W_hh, bias, ln_gamma, ln_beta, ln_c_gamma, ln_c_beta):
    B, I = x.shape
    H = c.shape[1]
    BB = min(512, B)
    nb = B // BB

    b2 = bias.reshape(1, 4 * H)
    g2 = ln_gamma.reshape(1, 4 * H)
    be2 = ln_beta.reshape(1, 4 * H)
    gc2 = ln_c_gamma.reshape(1, H)
    bc2 = ln_c_beta.reshape(1, H)

    row = lambda i: (i, 0)
    fixed = lambda i: (0, 0)
    new_h, new_c = pl.pallas_call(
        _lstm_kernel,
        grid=(nb,),
        in_specs=[
            pl.BlockSpec((BB, I), row),
            pl.BlockSpec((BB, H), row),
            pl.BlockSpec((BB, H), row),
            pl.BlockSpec((I, 4 * H), fixed),
            pl.BlockSpec((H, 4 * H), fixed),
            pl.BlockSpec((1, 4 * H), fixed),
            pl.BlockSpec((1, 4 * H), fixed),
            pl.BlockSpec((1, 4 * H), fixed),
            pl.BlockSpec((1, H), fixed),
            pl.BlockSpec((1, H), fixed),
        ],
        out_specs=[
            pl.BlockSpec((BB, H), row),
            pl.BlockSpec((BB, H), row),
        ],
        out_shape=[
            jax.ShapeDtypeStruct((B, H), jnp.float32),
            jax.ShapeDtypeStruct((B, H), jnp.float32),
        ],
        compiler_params=pltpu.CompilerParams(
            dimension_semantics=("parallel",),
            vmem_limit_bytes=100 * 1024 * 1024,
        ),
        name="lstm_cell_fused",
    )(x, c, h, W_xh, W_hh, b2, g2, be2, gc2, bc2)
    return new_h, new_c
